# baseline (speedup 1.0000x reference)
import functools

import jax
import jax.numpy as jnp
from jax import lax
from jax.experimental import pallas as pl
from jax.experimental.pallas import tpu as pltpu
from jax.experimental.pallas import tpu_sc as plsc

B = 16384
V = 1000000
D = 32
H1 = 256
H2 = 64

_info = plsc.get_sparse_core_info()
_NC, _NS = _info.num_cores, _info.num_subcores
_NW = _NC * _NS
_BPW = B // _NW
_TCHUNK = 64
_L = 16


def _gather_body(uid_hbm, mid_hbm, utab_hbm, ctab_hbm, uout_hbm, cout_hbm,
                 uidx_v, midx_v, ufi_v, mfi_v, uslot_v, cslot_v, sem):
    wid = lax.axis_index("s") * _NC + lax.axis_index("c")
    base = wid * _BPW
    pltpu.sync_copy(uid_hbm.at[pl.ds(base, _BPW)], uidx_v)
    pltpu.sync_copy(mid_hbm.at[pl.ds(base, _BPW)], midx_v)

    for r in range(D):
        for k0 in range(0, _BPW, _L):
            sl = pl.ds(k0, _L)
            ufi_v[r, sl] = uidx_v[sl] + (r * V)
            mfi_v[r, sl] = midx_v[sl] + (r * V)

    cps = []
    for r in range(D):
        cps.append(pltpu.async_copy(utab_hbm.at[ufi_v.at[r]],
                                    uslot_v.at[r], sem))
        cps.append(pltpu.async_copy(ctab_hbm.at[mfi_v.at[r]],
                                    cslot_v.at[r], sem))
    for cp in cps:
        cp.wait()

    cps = []
    for r in range(D):
        cps.append(pltpu.async_copy(uslot_v.at[r],
                                    uout_hbm.at[pl.ds(r * B + base, _BPW)],
                                    sem))
        cps.append(pltpu.async_copy(cslot_v.at[r],
                                    cout_hbm.at[pl.ds(r * B + base, _BPW)],
                                    sem))
    for cp in cps:
        cp.wait()


_gather = functools.partial(
    pl.kernel,
    mesh=plsc.VectorSubcoreMesh(core_axis_name="c", subcore_axis_name="s"),
    out_type=[
        jax.ShapeDtypeStruct((D * B,), jnp.float32),
        jax.ShapeDtypeStruct((D * B,), jnp.float32),
    ],
    scratch_types=[
        pltpu.VMEM((_BPW,), jnp.int32),
        pltpu.VMEM((_BPW,), jnp.int32),
        pltpu.VMEM((D, _BPW), jnp.int32),
        pltpu.VMEM((D, _BPW), jnp.int32),
        pltpu.VMEM((D, _BPW), jnp.float32),
        pltpu.VMEM((D, _BPW), jnp.float32),
        pltpu.SemaphoreType.DMA,
    ],
    compiler_params=pltpu.CompilerParams(needs_layout_passes=False,
                                         use_tc_tiling_on_sc=False),
)(_gather_body)


def _mlp_body(u_ref, c_ref, w1_ref, b1_ref, w2_ref, b2_ref, w3_ref, b3_ref,
              out_ref):
    uT = u_ref[...]
    cT = c_ref[...]
    ct = (((0,), (0,)), ((), ()))
    h = lax.dot_general(w1_ref[0:D, :], uT, ct,
                        preferred_element_type=jnp.float32)
    h += lax.dot_general(w1_ref[D:2 * D, :], cT, ct,
                         preferred_element_type=jnp.float32)
    h = jnp.maximum(h + b1_ref[...], 0.0)
    h = lax.dot_general(w2_ref[...], h, ct,
                        preferred_element_type=jnp.float32)
    h = jnp.maximum(h + b2_ref[...], 0.0)
    out_ref[...] = lax.dot_general(w3_ref[...], h, ct,
                                   preferred_element_type=jnp.float32) \
        + b3_ref[...]


def _mlp(uT, cT, W1, b1, W2, b2, W3, b3):
    BM = 2048
    grid = (B // BM,)
    return pl.pallas_call(
        _mlp_body,
        grid=grid,
        in_specs=[
            pl.BlockSpec((D, BM), lambda i: (0, i)),
            pl.BlockSpec((D, BM), lambda i: (0, i)),
            pl.BlockSpec((2 * D, H1), lambda i: (0, 0)),
            pl.BlockSpec((H1, 1), lambda i: (0, 0)),
            pl.BlockSpec((H1, H2), lambda i: (0, 0)),
            pl.BlockSpec((H2, 1), lambda i: (0, 0)),
            pl.BlockSpec((H2, 1), lambda i: (0, 0)),
            pl.BlockSpec((1, 1), lambda i: (0, 0)),
        ],
        out_specs=pl.BlockSpec((1, BM), lambda i: (0, i)),
        out_shape=jax.ShapeDtypeStruct((1, B), jnp.float32),
    )(uT, cT, W1, b1, W2, b2, W3, b3)


def kernel(user_id, movie_id, user_table, cand_table, W1, b1, W2, b2, W3, b3):
    utabf = user_table.T.reshape(D * V)
    ctabf = cand_table.T.reshape(D * V)
    uT3, cT3 = _gather(user_id.astype(jnp.int32), movie_id.astype(jnp.int32),
                       utabf, ctabf)
    uT = uT3.reshape(D, B)
    cT = cT3.reshape(D, B)
    outT = _mlp(uT, cT, W1, b1.reshape(H1, 1), W2, b2.reshape(H2, 1),
                W3, b3.reshape(1, 1))
    return outT.reshape(B, 1)


# speedup vs baseline: 11.8204x; 11.8204x over previous
import functools

import jax
import jax.numpy as jnp
from jax import lax
from jax.experimental import pallas as pl
from jax.experimental.pallas import tpu as pltpu
from jax.experimental.pallas import tpu_sc as plsc

B = 16384
V = 1000000
D = 32
H1 = 256
H2 = 64

_info = plsc.get_sparse_core_info()
_NC, _NS = _info.num_cores, _info.num_subcores
_NW = _NC * _NS
_BPW = B // _NW
_TCHUNK = 64
_L = 16


_VCOV = (V // 128) * 128
_WC = 16384
_NCH = _VCOV // _WC


def _detile_body(utab_hbm, ctab_hbm, uout_hbm, cout_hbm, slab_v, sem):
    wid = lax.axis_index("s") * _NC + lax.axis_index("c")

    for r in range(D):
        @pl.when(wid == r)
        def _():
            def chunk(c0, width, buf):
                pltpu.sync_copy(utab_hbm.at[r, pl.ds(c0, width)], buf)
                pltpu.sync_copy(buf, uout_hbm.at[pl.ds(r * V + c0, width)])
                pltpu.sync_copy(ctab_hbm.at[r, pl.ds(c0, width)], buf)
                pltpu.sync_copy(buf, cout_hbm.at[pl.ds(r * V + c0, width)])

            def loop_body(i, _):
                chunk(pl.multiple_of(i * _WC, 128), _WC, slab_v)
                return ()

            lax.fori_loop(0, _NCH, loop_body, ())
            chunk(_NCH * _WC, _VCOV - _NCH * _WC, slab_v.at[pl.ds(0, 512)])


_detile = functools.partial(
    pl.kernel,
    mesh=plsc.VectorSubcoreMesh(core_axis_name="c", subcore_axis_name="s"),
    out_type=[
        jax.ShapeDtypeStruct((D * V,), jnp.float32),
        jax.ShapeDtypeStruct((D * V,), jnp.float32),
    ],
    scratch_types=[
        pltpu.VMEM((_WC,), jnp.float32),
        pltpu.SemaphoreType.DMA,
    ],
    compiler_params=pltpu.CompilerParams(needs_layout_passes=False,
                                         use_tc_tiling_on_sc=True),
)(_detile_body)


def _gather_body(uid_hbm, mid_hbm, utab_hbm, ctab_hbm, uout_hbm, cout_hbm,
                 uidx_v, midx_v, ufi_v, mfi_v, uslot_v, cslot_v, sem):
    wid = lax.axis_index("s") * _NC + lax.axis_index("c")
    base = wid * _BPW
    pltpu.sync_copy(uid_hbm.at[pl.ds(base, _BPW)], uidx_v)
    pltpu.sync_copy(mid_hbm.at[pl.ds(base, _BPW)], midx_v)

    for r in range(D):
        for k0 in range(0, _BPW, _L):
            sl = pl.ds(k0, _L)
            ufi_v[r, sl] = uidx_v[sl] + (r * V)
            mfi_v[r, sl] = midx_v[sl] + (r * V)

    cps = []
    for r in range(D):
        cps.append(pltpu.async_copy(utab_hbm.at[ufi_v.at[r]],
                                    uslot_v.at[r], sem))
        cps.append(pltpu.async_copy(ctab_hbm.at[mfi_v.at[r]],
                                    cslot_v.at[r], sem))
    for cp in cps:
        cp.wait()

    cps = []
    for r in range(D):
        cps.append(pltpu.async_copy(uslot_v.at[r],
                                    uout_hbm.at[pl.ds(r * B + base, _BPW)],
                                    sem))
        cps.append(pltpu.async_copy(cslot_v.at[r],
                                    cout_hbm.at[pl.ds(r * B + base, _BPW)],
                                    sem))
    for cp in cps:
        cp.wait()


_gather = functools.partial(
    pl.kernel,
    mesh=plsc.VectorSubcoreMesh(core_axis_name="c", subcore_axis_name="s"),
    out_type=[
        jax.ShapeDtypeStruct((D * B,), jnp.float32),
        jax.ShapeDtypeStruct((D * B,), jnp.float32),
    ],
    scratch_types=[
        pltpu.VMEM((_BPW,), jnp.int32),
        pltpu.VMEM((_BPW,), jnp.int32),
        pltpu.VMEM((D, _BPW), jnp.int32),
        pltpu.VMEM((D, _BPW), jnp.int32),
        pltpu.VMEM((D, _BPW), jnp.float32),
        pltpu.VMEM((D, _BPW), jnp.float32),
        pltpu.SemaphoreType.DMA,
    ],
    compiler_params=pltpu.CompilerParams(needs_layout_passes=False,
                                         use_tc_tiling_on_sc=False),
)(_gather_body)


def _mlp_body(u_ref, c_ref, w1_ref, b1_ref, w2_ref, b2_ref, w3_ref, b3_ref,
              out_ref):
    uT = u_ref[...]
    cT = c_ref[...]
    ct = (((0,), (0,)), ((), ()))
    h = lax.dot_general(w1_ref[0:D, :], uT, ct,
                        preferred_element_type=jnp.float32)
    h += lax.dot_general(w1_ref[D:2 * D, :], cT, ct,
                         preferred_element_type=jnp.float32)
    h = jnp.maximum(h + b1_ref[...], 0.0)
    h = lax.dot_general(w2_ref[...], h, ct,
                        preferred_element_type=jnp.float32)
    h = jnp.maximum(h + b2_ref[...], 0.0)
    out_ref[...] = lax.dot_general(w3_ref[...], h, ct,
                                   preferred_element_type=jnp.float32) \
        + b3_ref[...]


def _mlp(uT, cT, W1, b1, W2, b2, W3, b3):
    BM = 2048
    grid = (B // BM,)
    return pl.pallas_call(
        _mlp_body,
        grid=grid,
        in_specs=[
            pl.BlockSpec((D, BM), lambda i: (0, i)),
            pl.BlockSpec((D, BM), lambda i: (0, i)),
            pl.BlockSpec((2 * D, H1), lambda i: (0, 0)),
            pl.BlockSpec((H1, 1), lambda i: (0, 0)),
            pl.BlockSpec((H1, H2), lambda i: (0, 0)),
            pl.BlockSpec((H2, 1), lambda i: (0, 0)),
            pl.BlockSpec((H2, 1), lambda i: (0, 0)),
            pl.BlockSpec((1, 1), lambda i: (0, 0)),
        ],
        out_specs=pl.BlockSpec((1, BM), lambda i: (0, i)),
        out_shape=jax.ShapeDtypeStruct((1, B), jnp.float32),
    )(uT, cT, W1, b1, W2, b2, W3, b3)


def kernel(user_id, movie_id, user_table, cand_table, W1, b1, W2, b2, W3, b3):
    utabf, ctabf = _detile(user_table.T, cand_table.T)
    uT3, cT3 = _gather(user_id.astype(jnp.int32), movie_id.astype(jnp.int32),
                       utabf, ctabf)
    uid32 = user_id.astype(jnp.int32)
    mid32 = movie_id.astype(jnp.int32)
    uT = uT3.reshape(D, B)
    cT = cT3.reshape(D, B)
    tail_u = user_table[_VCOV:].T
    tail_c = cand_table[_VCOV:].T
    um = uid32 >= _VCOV
    cm = mid32 >= _VCOV
    uT = jnp.where(um[None, :], tail_u[:, jnp.clip(uid32 - _VCOV, 0, 63)], uT)
    cT = jnp.where(cm[None, :], tail_c[:, jnp.clip(mid32 - _VCOV, 0, 63)], cT)
    outT = _mlp(uT, cT, W1, b1.reshape(H1, 1), W2, b2.reshape(H2, 1),
                W3, b3.reshape(1, 1))
    return outT.reshape(B, 1)


# speedup vs baseline: 14.2441x; 1.2050x over previous
import functools

import jax
import jax.numpy as jnp
from jax import lax
from jax.experimental import pallas as pl
from jax.experimental.pallas import tpu as pltpu
from jax.experimental.pallas import tpu_sc as plsc

B = 16384
V = 1000000
D = 32
H1 = 256
H2 = 64

_info = plsc.get_sparse_core_info()
_NC, _NS = _info.num_cores, _info.num_subcores
_NW = _NC * _NS
_BPW = B // _NW
_TCHUNK = 64
_L = 16


_VCOV = (V // 128) * 128
_WC = 32768
_NCH = _VCOV // _WC
_TW = _VCOV - _NCH * _WC


def _detile_body(utab_hbm, ctab_hbm, uout_hbm, cout_hbm, slab_u, slab_c, sem):
    wid = lax.axis_index("s") * _NC + lax.axis_index("c")

    for r in range(D):
        @pl.when(wid == r)
        def _():
            def chunk(c0, width, bu, bc):
                cpu = pltpu.async_copy(utab_hbm.at[r, pl.ds(c0, width)],
                                       bu, sem)
                cpc = pltpu.async_copy(ctab_hbm.at[r, pl.ds(c0, width)],
                                       bc, sem)
                cpu.wait()
                cpc.wait()
                ou = pltpu.async_copy(bu, uout_hbm.at[pl.ds(r * V + c0,
                                                            width)], sem)
                oc = pltpu.async_copy(bc, cout_hbm.at[pl.ds(r * V + c0,
                                                            width)], sem)
                ou.wait()
                oc.wait()

            def loop_body(i, _):
                chunk(pl.multiple_of(i * _WC, 128), _WC, slab_u, slab_c)
                return ()

            lax.fori_loop(0, _NCH, loop_body, ())
            chunk(_NCH * _WC, _VCOV - _NCH * _WC,
                  slab_u.at[pl.ds(0, _TW)], slab_c.at[pl.ds(0, _TW)])


_detile = functools.partial(
    pl.kernel,
    mesh=plsc.VectorSubcoreMesh(core_axis_name="c", subcore_axis_name="s"),
    out_type=[
        jax.ShapeDtypeStruct((D * V,), jnp.float32),
        jax.ShapeDtypeStruct((D * V,), jnp.float32),
    ],
    scratch_types=[
        pltpu.VMEM((_WC,), jnp.float32),
        pltpu.VMEM((_WC,), jnp.float32),
        pltpu.SemaphoreType.DMA,
    ],
    compiler_params=pltpu.CompilerParams(needs_layout_passes=False,
                                         use_tc_tiling_on_sc=True),
)(_detile_body)


def _gather_body(uid_hbm, mid_hbm, utab_hbm, ctab_hbm, uout_hbm, cout_hbm,
                 uidx_v, midx_v, ufi_v, mfi_v, uslot_v, cslot_v, sem):
    wid = lax.axis_index("s") * _NC + lax.axis_index("c")
    base = wid * _BPW
    pltpu.sync_copy(uid_hbm.at[pl.ds(base, _BPW)], uidx_v)
    pltpu.sync_copy(mid_hbm.at[pl.ds(base, _BPW)], midx_v)

    for r in range(D):
        for k0 in range(0, _BPW, _L):
            sl = pl.ds(k0, _L)
            ufi_v[r, sl] = uidx_v[sl] + (r * V)
            mfi_v[r, sl] = midx_v[sl] + (r * V)

    cps = []
    for r in range(D):
        cps.append(pltpu.async_copy(utab_hbm.at[ufi_v.at[r]],
                                    uslot_v.at[r], sem))
        cps.append(pltpu.async_copy(ctab_hbm.at[mfi_v.at[r]],
                                    cslot_v.at[r], sem))
    for cp in cps:
        cp.wait()

    cps = []
    for r in range(D):
        cps.append(pltpu.async_copy(uslot_v.at[r],
                                    uout_hbm.at[pl.ds(r * B + base, _BPW)],
                                    sem))
        cps.append(pltpu.async_copy(cslot_v.at[r],
                                    cout_hbm.at[pl.ds(r * B + base, _BPW)],
                                    sem))
    for cp in cps:
        cp.wait()


_gather = functools.partial(
    pl.kernel,
    mesh=plsc.VectorSubcoreMesh(core_axis_name="c", subcore_axis_name="s"),
    out_type=[
        jax.ShapeDtypeStruct((D * B,), jnp.float32),
        jax.ShapeDtypeStruct((D * B,), jnp.float32),
    ],
    scratch_types=[
        pltpu.VMEM((_BPW,), jnp.int32),
        pltpu.VMEM((_BPW,), jnp.int32),
        pltpu.VMEM((D, _BPW), jnp.int32),
        pltpu.VMEM((D, _BPW), jnp.int32),
        pltpu.VMEM((D, _BPW), jnp.float32),
        pltpu.VMEM((D, _BPW), jnp.float32),
        pltpu.SemaphoreType.DMA,
    ],
    compiler_params=pltpu.CompilerParams(needs_layout_passes=False,
                                         use_tc_tiling_on_sc=False),
)(_gather_body)


def _mlp_body(u_ref, c_ref, w1_ref, b1_ref, w2_ref, b2_ref, w3_ref, b3_ref,
              out_ref):
    uT = u_ref[...]
    cT = c_ref[...]
    ct = (((0,), (0,)), ((), ()))
    h = lax.dot_general(w1_ref[0:D, :], uT, ct,
                        preferred_element_type=jnp.float32)
    h += lax.dot_general(w1_ref[D:2 * D, :], cT, ct,
                         preferred_element_type=jnp.float32)
    h = jnp.maximum(h + b1_ref[...], 0.0)
    h = lax.dot_general(w2_ref[...], h, ct,
                        preferred_element_type=jnp.float32)
    h = jnp.maximum(h + b2_ref[...], 0.0)
    out_ref[...] = lax.dot_general(w3_ref[...], h, ct,
                                   preferred_element_type=jnp.float32) \
        + b3_ref[...]


def _mlp(uT, cT, W1, b1, W2, b2, W3, b3):
    BM = 2048
    grid = (B // BM,)
    return pl.pallas_call(
        _mlp_body,
        grid=grid,
        in_specs=[
            pl.BlockSpec((D, BM), lambda i: (0, i)),
            pl.BlockSpec((D, BM), lambda i: (0, i)),
            pl.BlockSpec((2 * D, H1), lambda i: (0, 0)),
            pl.BlockSpec((H1, 1), lambda i: (0, 0)),
            pl.BlockSpec((H1, H2), lambda i: (0, 0)),
            pl.BlockSpec((H2, 1), lambda i: (0, 0)),
            pl.BlockSpec((H2, 1), lambda i: (0, 0)),
            pl.BlockSpec((1, 1), lambda i: (0, 0)),
        ],
        out_specs=pl.BlockSpec((1, BM), lambda i: (0, i)),
        out_shape=jax.ShapeDtypeStruct((1, B), jnp.float32),
    )(uT, cT, W1, b1, W2, b2, W3, b3)


def kernel(user_id, movie_id, user_table, cand_table, W1, b1, W2, b2, W3, b3):
    utabf, ctabf = _detile(user_table.T, cand_table.T)
    uT3, cT3 = _gather(user_id.astype(jnp.int32), movie_id.astype(jnp.int32),
                       utabf, ctabf)
    uid32 = user_id.astype(jnp.int32)
    mid32 = movie_id.astype(jnp.int32)
    uT = uT3.reshape(D, B)
    cT = cT3.reshape(D, B)
    tail_u = user_table[_VCOV:].T
    tail_c = cand_table[_VCOV:].T
    um = uid32 >= _VCOV
    cm = mid32 >= _VCOV
    uT = jnp.where(um[None, :], tail_u[:, jnp.clip(uid32 - _VCOV, 0, 63)], uT)
    cT = jnp.where(cm[None, :], tail_c[:, jnp.clip(mid32 - _VCOV, 0, 63)], cT)
    outT = _mlp(uT, cT, W1, b1.reshape(H1, 1), W2, b2.reshape(H2, 1),
                W3, b3.reshape(1, 1))
    return outT.reshape(B, 1)


# speedup vs baseline: 14.6095x; 1.0257x over previous
import functools

import jax
import jax.numpy as jnp
from jax import lax
from jax.experimental import pallas as pl
from jax.experimental.pallas import tpu as pltpu
from jax.experimental.pallas import tpu_sc as plsc

B = 16384
V = 1000000
D = 32
H1 = 256
H2 = 64

_info = plsc.get_sparse_core_info()
_NC, _NS = _info.num_cores, _info.num_subcores
_NW = _NC * _NS
_BPW = B // _NW
_TCHUNK = 64
_L = 16


_VCOV = (V // 128) * 128
_WC = 24576
_NPAIR = _VCOV // (2 * _WC)
_TW = _VCOV - _NPAIR * 2 * _WC


def _detile_body(utab_hbm, ctab_hbm, uout_hbm, cout_hbm, slab_u, slab_c,
                 slab_u2, slab_c2, sem):
    wid = lax.axis_index("s") * _NC + lax.axis_index("c")

    for r in range(D):
        @pl.when(wid == r)
        def _():
            def fire_in(c0, width, bu, bc):
                cpu = pltpu.async_copy(utab_hbm.at[r, pl.ds(c0, width)],
                                       bu, sem)
                cpc = pltpu.async_copy(ctab_hbm.at[r, pl.ds(c0, width)],
                                       bc, sem)
                return cpu, cpc

            def fire_out(c0, width, bu, bc):
                ou = pltpu.async_copy(bu, uout_hbm.at[pl.ds(r * V + c0,
                                                            width)], sem)
                oc = pltpu.async_copy(bc, cout_hbm.at[pl.ds(r * V + c0,
                                                            width)], sem)
                return ou, oc

            def loop_body(i, _):
                c0 = pl.multiple_of(i * 2 * _WC, 128)
                c1 = c0 + _WC
                iu0, ic0 = fire_in(c0, _WC, slab_u, slab_c)
                iu1, ic1 = fire_in(c1, _WC, slab_u2, slab_c2)
                iu0.wait()
                ic0.wait()
                ou0, oc0 = fire_out(c0, _WC, slab_u, slab_c)
                iu1.wait()
                ic1.wait()
                ou1, oc1 = fire_out(c1, _WC, slab_u2, slab_c2)
                ou0.wait()
                oc0.wait()
                ou1.wait()
                oc1.wait()
                return ()

            lax.fori_loop(0, _NPAIR, loop_body, ())
            t0 = _NPAIR * 2 * _WC
            iu0, ic0 = fire_in(t0, _TW, slab_u.at[pl.ds(0, _TW)],
                               slab_c.at[pl.ds(0, _TW)])
            iu0.wait()
            ic0.wait()
            ou0, oc0 = fire_out(t0, _TW, slab_u.at[pl.ds(0, _TW)],
                                slab_c.at[pl.ds(0, _TW)])
            ou0.wait()
            oc0.wait()


_detile = functools.partial(
    pl.kernel,
    mesh=plsc.VectorSubcoreMesh(core_axis_name="c", subcore_axis_name="s"),
    out_type=[
        jax.ShapeDtypeStruct((D * V,), jnp.float32),
        jax.ShapeDtypeStruct((D * V,), jnp.float32),
    ],
    scratch_types=[
        pltpu.VMEM((_WC,), jnp.float32),
        pltpu.VMEM((_WC,), jnp.float32),
        pltpu.VMEM((_WC,), jnp.float32),
        pltpu.VMEM((_WC,), jnp.float32),
        pltpu.SemaphoreType.DMA,
    ],
    compiler_params=pltpu.CompilerParams(needs_layout_passes=False,
                                         use_tc_tiling_on_sc=True),
)(_detile_body)


def _gather_body(uid_hbm, mid_hbm, utab_hbm, ctab_hbm, uout_hbm, cout_hbm,
                 uidx_v, midx_v, ufi_v, mfi_v, uslot_v, cslot_v, sem):
    wid = lax.axis_index("s") * _NC + lax.axis_index("c")
    base = wid * _BPW
    pltpu.sync_copy(uid_hbm.at[pl.ds(base, _BPW)], uidx_v)
    pltpu.sync_copy(mid_hbm.at[pl.ds(base, _BPW)], midx_v)

    for r in range(D):
        for k0 in range(0, _BPW, _L):
            sl = pl.ds(k0, _L)
            ufi_v[r, sl] = uidx_v[sl] + (r * V)
            mfi_v[r, sl] = midx_v[sl] + (r * V)

    cps = []
    for r in range(D):
        cps.append(pltpu.async_copy(utab_hbm.at[ufi_v.at[r]],
                                    uslot_v.at[r], sem))
        cps.append(pltpu.async_copy(ctab_hbm.at[mfi_v.at[r]],
                                    cslot_v.at[r], sem))
    for cp in cps:
        cp.wait()

    cps = []
    for r in range(D):
        cps.append(pltpu.async_copy(uslot_v.at[r],
                                    uout_hbm.at[pl.ds(r * B + base, _BPW)],
                                    sem))
        cps.append(pltpu.async_copy(cslot_v.at[r],
                                    cout_hbm.at[pl.ds(r * B + base, _BPW)],
                                    sem))
    for cp in cps:
        cp.wait()


_gather = functools.partial(
    pl.kernel,
    mesh=plsc.VectorSubcoreMesh(core_axis_name="c", subcore_axis_name="s"),
    out_type=[
        jax.ShapeDtypeStruct((D * B,), jnp.float32),
        jax.ShapeDtypeStruct((D * B,), jnp.float32),
    ],
    scratch_types=[
        pltpu.VMEM((_BPW,), jnp.int32),
        pltpu.VMEM((_BPW,), jnp.int32),
        pltpu.VMEM((D, _BPW), jnp.int32),
        pltpu.VMEM((D, _BPW), jnp.int32),
        pltpu.VMEM((D, _BPW), jnp.float32),
        pltpu.VMEM((D, _BPW), jnp.float32),
        pltpu.SemaphoreType.DMA,
    ],
    compiler_params=pltpu.CompilerParams(needs_layout_passes=False,
                                         use_tc_tiling_on_sc=False),
)(_gather_body)


def _mlp_body(u_ref, c_ref, w1_ref, b1_ref, w2_ref, b2_ref, w3_ref, b3_ref,
              out_ref):
    uT = u_ref[...]
    cT = c_ref[...]
    ct = (((0,), (0,)), ((), ()))
    h = lax.dot_general(w1_ref[0:D, :], uT, ct,
                        preferred_element_type=jnp.float32)
    h += lax.dot_general(w1_ref[D:2 * D, :], cT, ct,
                         preferred_element_type=jnp.float32)
    h = jnp.maximum(h + b1_ref[...], 0.0)
    h = lax.dot_general(w2_ref[...], h, ct,
                        preferred_element_type=jnp.float32)
    h = jnp.maximum(h + b2_ref[...], 0.0)
    out_ref[...] = lax.dot_general(w3_ref[...], h, ct,
                                   preferred_element_type=jnp.float32) \
        + b3_ref[...]


def _mlp(uT, cT, W1, b1, W2, b2, W3, b3):
    BM = 2048
    grid = (B // BM,)
    return pl.pallas_call(
        _mlp_body,
        grid=grid,
        in_specs=[
            pl.BlockSpec((D, BM), lambda i: (0, i)),
            pl.BlockSpec((D, BM), lambda i: (0, i)),
            pl.BlockSpec((2 * D, H1), lambda i: (0, 0)),
            pl.BlockSpec((H1, 1), lambda i: (0, 0)),
            pl.BlockSpec((H1, H2), lambda i: (0, 0)),
            pl.BlockSpec((H2, 1), lambda i: (0, 0)),
            pl.BlockSpec((H2, 1), lambda i: (0, 0)),
            pl.BlockSpec((1, 1), lambda i: (0, 0)),
        ],
        out_specs=pl.BlockSpec((1, BM), lambda i: (0, i)),
        out_shape=jax.ShapeDtypeStruct((1, B), jnp.float32),
    )(uT, cT, W1, b1, W2, b2, W3, b3)


def kernel(user_id, movie_id, user_table, cand_table, W1, b1, W2, b2, W3, b3):
    utabf, ctabf = _detile(user_table.T, cand_table.T)
    uT3, cT3 = _gather(user_id.astype(jnp.int32), movie_id.astype(jnp.int32),
                       utabf, ctabf)
    uid32 = user_id.astype(jnp.int32)
    mid32 = movie_id.astype(jnp.int32)
    uT = uT3.reshape(D, B)
    cT = cT3.reshape(D, B)
    tail_u = user_table[_VCOV:].T
    tail_c = cand_table[_VCOV:].T
    um = uid32 >= _VCOV
    cm = mid32 >= _VCOV
    uT = jnp.where(um[None, :], tail_u[:, jnp.clip(uid32 - _VCOV, 0, 63)], uT)
    cT = jnp.where(cm[None, :], tail_c[:, jnp.clip(mid32 - _VCOV, 0, 63)], cT)
    outT = _mlp(uT, cT, W1, b1.reshape(H1, 1), W2, b2.reshape(H2, 1),
                W3, b3.reshape(1, 1))
    return outT.reshape(B, 1)
